# baseline (device time: 29229 ns/iter reference)
import jax
import jax.numpy as jnp
from jax import lax
from jax.experimental import pallas as pl
from jax.experimental.pallas import tpu as pltpu

B, S, H, Dh, Dr, D = 2, 256, 16, 64, 32, 1024
T = B * S
DC = 64
HH = H // 2
HD = HH * Dh
SCALE = (Dh + Dr) ** -0.5
MESH = pl.DeviceIdType.MESH
BF = jnp.bfloat16
F32 = jnp.float32


def _dot(a, b):
    return jnp.dot(a, b, preferred_element_type=F32)


def kernel(x, Wdkv, Wuk, Wuv, Wq, Wqr, Wkr, Wo):
    def body(x_ref, wdkv_ref, wuk_ref, wuv_ref, wkr_ref,
             wq_hbm, wqr_ref, wo_hbm, out_ref,
             wq_v, wo_v, c_ref, c_recv, wuk_recv, wuv_recv,
             o16_ref, out_v, copy_sems, send_sems, recv_sems,
             o_send_sems, o_recv_sems, out_sems):
        my_x = lax.axis_index("x")
        my_y = lax.axis_index("y")
        my_z = lax.axis_index("z")
        xpartner = (1 - my_x, my_y, my_z)
        ypartner = (my_x, 1 - my_y, my_z)

        def make_wq_cp(hy):
            hc0 = hy * HD
            return pltpu.make_async_copy(
                wq_hbm.at[:, hc0:hc0 + HD], wq_v, copy_sems.at[0])

        @pl.when(my_y == 0)
        def _():
            make_wq_cp(0).start()

        @pl.when(my_y == 1)
        def _():
            make_wq_cp(1).start()

        wo_cp = pltpu.make_async_copy(wo_hbm, wo_v, copy_sems.at[1])
        wo_cp.start()

        barrier_sem = pltpu.get_barrier_semaphore()
        pl.semaphore_signal(barrier_sem, inc=1, device_id=xpartner,
                            device_id_type=MESH)
        pl.semaphore_signal(barrier_sem, inc=1, device_id=ypartner,
                            device_id_type=MESH)

        xf = x_ref[...].reshape(T, D)
        c_ref[...] = _dot(xf, wdkv_ref[...]).astype(BF)

        def run_half(hy):
            hc0 = hy * HD

            pl.semaphore_wait(barrier_sem, 2)
            rdmas = []
            for i, (src, dst) in enumerate(
                    [(c_ref, c_recv),
                     (wuk_ref.at[:, hc0:hc0 + HD], wuk_recv),
                     (wuv_ref.at[:, hc0:hc0 + HD], wuv_recv)]):
                r = pltpu.make_async_remote_copy(
                    src_ref=src, dst_ref=dst,
                    send_sem=send_sems.at[i], recv_sem=recv_sems.at[i],
                    device_id=xpartner, device_id_type=MESH,
                )
                r.start()
                rdmas.append(r)

            xq = xf * jnp.asarray(SCALE, BF)
            make_wq_cp(hy).wait()
            q_all = _dot(xq, wq_v[...])
            rc0 = hy * HH * Dr
            qr_all = _dot(xq, wqr_ref[:, rc0:rc0 + HH * Dr])
            kr_all = _dot(xf, wkr_ref[...])

            for r in rdmas:
                r.wait()
            c_loc = c_ref[...]
            c_rem = c_recv[...]
            wuk_loc = wuk_ref[:, hc0:hc0 + HD]
            wuv_loc = wuv_ref[:, hc0:hc0 + HD]
            k_all = _dot(c_loc, wuk_loc) + _dot(c_rem, wuk_recv[...])
            v_all = _dot(c_loc, wuv_loc) + _dot(c_rem, wuv_recv[...])

            o_rdmas = []
            for b in range(B):
                kr_b = kr_all[b * S:(b + 1) * S, :].astype(BF)
                for i in range(HH):
                    r0, r1 = b * S, (b + 1) * S
                    lc0, lc1 = i * Dh, (i + 1) * Dh
                    gc0 = hc0 + lc0
                    q = q_all[r0:r1, lc0:lc1].astype(BF)
                    k = k_all[r0:r1, lc0:lc1].astype(BF)
                    v = v_all[r0:r1, lc0:lc1].astype(BF)
                    qr = qr_all[r0:r1, i * Dr:(i + 1) * Dr].astype(BF)
                    s = (lax.dot_general(q, k, (((1,), (1,)), ((), ())),
                                         preferred_element_type=F32)
                         + lax.dot_general(qr, kr_b,
                                           (((1,), (1,)), ((), ())),
                                           preferred_element_type=F32))
                    e = jnp.exp(s.astype(BF))
                    denom = jnp.sum(e, axis=-1, keepdims=True, dtype=F32)
                    o16_ref[r0:r1, gc0:gc0 + Dh] = (_dot(e, v)
                                                    / denom).astype(BF)
                    if i % 2 == 1:
                        jj = b * (HH // 2) + i // 2
                        pc0 = gc0 - Dh
                        r = pltpu.make_async_remote_copy(
                            src_ref=o16_ref.at[r0:r1, pc0:pc0 + 2 * Dh],
                            dst_ref=o16_ref.at[r0:r1, pc0:pc0 + 2 * Dh],
                            send_sem=o_send_sems.at[jj],
                            recv_sem=o_recv_sems.at[jj],
                            device_id=ypartner, device_id_type=MESH,
                        )
                        r.start()
                        o_rdmas.append(r)

            for r in o_rdmas:
                r.wait()

        @pl.when(my_y == 0)
        def _():
            run_half(0)

        @pl.when(my_y == 1)
        def _():
            run_half(1)

        wo_cp.wait()
        out_cps = []
        for b in range(B):
            out_v[b] = _dot(o16_ref[b * S:(b + 1) * S, :],
                            wo_v[...]).astype(BF)
            cp = pltpu.make_async_copy(out_v.at[b], out_ref.at[b],
                                       out_sems.at[b])
            cp.start()
            out_cps.append(cp)
        for cp in out_cps:
            cp.wait()

    call = pl.pallas_call(
        body,
        out_shape=jax.ShapeDtypeStruct((B, S, D), BF),
        in_specs=[
            pl.BlockSpec(memory_space=pltpu.VMEM),
            pl.BlockSpec(memory_space=pltpu.VMEM),
            pl.BlockSpec(memory_space=pltpu.VMEM),
            pl.BlockSpec(memory_space=pltpu.VMEM),
            pl.BlockSpec(memory_space=pltpu.VMEM),
            pl.BlockSpec(memory_space=pltpu.HBM),
            pl.BlockSpec(memory_space=pltpu.VMEM),
            pl.BlockSpec(memory_space=pltpu.HBM),
        ],
        out_specs=pl.BlockSpec(memory_space=pltpu.HBM),
        scratch_shapes=[
            pltpu.VMEM((D, HD), BF),
            pltpu.VMEM((D, D), BF),
            pltpu.VMEM((T, DC), BF),
            pltpu.VMEM((T, DC), BF),
            pltpu.VMEM((DC, HD), BF),
            pltpu.VMEM((DC, HD), BF),
            pltpu.VMEM((T, D), BF),
            pltpu.VMEM((B, S, D), BF),
            pltpu.SemaphoreType.DMA((2,)),
            pltpu.SemaphoreType.DMA((3,)),
            pltpu.SemaphoreType.DMA((3,)),
            pltpu.SemaphoreType.DMA((HH,)),
            pltpu.SemaphoreType.DMA((HH,)),
            pltpu.SemaphoreType.DMA((B,)),
        ],
        compiler_params=pltpu.CompilerParams(collective_id=0),
    )
    return call(x.astype(BF), Wdkv.astype(BF), Wuk.astype(BF),
                Wuv.astype(BF), Wkr.astype(BF), Wq.astype(BF),
                Wqr.astype(BF), Wo.astype(BF))


# device time: 28759 ns/iter; 1.0163x vs baseline; 1.0163x over previous
import jax
import jax.numpy as jnp
from jax import lax
from jax.experimental import pallas as pl
from jax.experimental.pallas import tpu as pltpu

B, S, H, Dh, Dr, D = 2, 256, 16, 64, 32, 1024
T = B * S
DC = 64
HH = H // 2
HD = HH * Dh
SCALE = (Dh + Dr) ** -0.5
MESH = pl.DeviceIdType.MESH
BF = jnp.bfloat16
F32 = jnp.float32


def _dot(a, b):
    return jnp.dot(a, b, preferred_element_type=F32)


def kernel(x, Wdkv, Wuk, Wuv, Wq, Wqr, Wkr, Wo):
    def body(x_ref, wdkr_ref, wukv_ref,
             wq_ref, wqr_ref, wo_ref, out_ref,
             c_ref, c_recv, wukv_recv,
             o16_ref, out_v, send_sems, recv_sems,
             o_send_sems, o_recv_sems, out_sems):
        my_x = lax.axis_index("x")
        my_y = lax.axis_index("y")
        my_z = lax.axis_index("z")
        xpartner = (1 - my_x, my_y, my_z)
        ypartner = (my_x, 1 - my_y, my_z)

        barrier_sem = pltpu.get_barrier_semaphore()
        pl.semaphore_signal(barrier_sem, inc=1, device_id=xpartner,
                            device_id_type=MESH)
        pl.semaphore_signal(barrier_sem, inc=1, device_id=ypartner,
                            device_id_type=MESH)

        xf = x_ref[...].reshape(T, D)
        wdkr = wdkr_ref[...]
        c_ref[...] = _dot(xf, wdkr[:, :DC]).astype(BF)

        def run_half(hy):
            hc0 = hy * HD

            pl.semaphore_wait(barrier_sem, 2)
            rdmas = []
            for i, (src, dst) in enumerate(
                    [(c_ref, c_recv),
                     (wukv_ref.at[:, hc0:hc0 + HD], wukv_recv)]):
                r = pltpu.make_async_remote_copy(
                    src_ref=src, dst_ref=dst,
                    send_sem=send_sems.at[i], recv_sem=recv_sems.at[i],
                    device_id=xpartner, device_id_type=MESH,
                )
                r.start()
                rdmas.append(r)

            xq = xf * jnp.asarray(SCALE, BF)
            q_all = _dot(xq, wq_ref[:, hc0:hc0 + HD])
            rc0 = hy * HH * Dr
            qr_all = _dot(xq, wqr_ref[:, rc0:rc0 + HH * Dr])
            kr_all = _dot(xf, wdkr[:, DC:])

            for r in rdmas:
                r.wait()
            c_loc = c_ref[...]
            c_rem = c_recv[...]
            wukv_loc = wukv_ref[:, hc0:hc0 + HD]
            wukv_rem = wukv_recv[...]
            k_all = (_dot(c_loc, wukv_loc[:DC])
                     + _dot(c_rem, wukv_rem[:DC]))
            v_all = (_dot(c_loc, wukv_loc[DC:])
                     + _dot(c_rem, wukv_rem[DC:]))

            o_rdmas = []
            for b in range(B):
                kr_b = kr_all[b * S:(b + 1) * S, :].astype(BF)
                for i in range(HH):
                    r0, r1 = b * S, (b + 1) * S
                    lc0, lc1 = i * Dh, (i + 1) * Dh
                    gc0 = hc0 + lc0
                    q = q_all[r0:r1, lc0:lc1].astype(BF)
                    k = k_all[r0:r1, lc0:lc1].astype(BF)
                    v = v_all[r0:r1, lc0:lc1].astype(BF)
                    qr = qr_all[r0:r1, i * Dr:(i + 1) * Dr].astype(BF)
                    s = (lax.dot_general(q, k, (((1,), (1,)), ((), ())),
                                         preferred_element_type=F32)
                         + lax.dot_general(qr, kr_b,
                                           (((1,), (1,)), ((), ())),
                                           preferred_element_type=F32))
                    e = jnp.exp(s.astype(BF))
                    denom = jnp.sum(e, axis=-1, keepdims=True, dtype=F32)
                    o16_ref[r0:r1, gc0:gc0 + Dh] = (_dot(e, v)
                                                    / denom).astype(BF)
                    if i % 2 == 1:
                        jj = b * (HH // 2) + i // 2
                        pc0 = gc0 - Dh
                        r = pltpu.make_async_remote_copy(
                            src_ref=o16_ref.at[r0:r1, pc0:pc0 + 2 * Dh],
                            dst_ref=o16_ref.at[r0:r1, pc0:pc0 + 2 * Dh],
                            send_sem=o_send_sems.at[jj],
                            recv_sem=o_recv_sems.at[jj],
                            device_id=ypartner, device_id_type=MESH,
                        )
                        r.start()
                        o_rdmas.append(r)

            for r in o_rdmas:
                r.wait()

        @pl.when(my_y == 0)
        def _():
            run_half(0)

        @pl.when(my_y == 1)
        def _():
            run_half(1)

        out_cps = []
        for b in range(B):
            out_v[b] = _dot(o16_ref[b * S:(b + 1) * S, :],
                            wo_ref[...]).astype(BF)
            cp = pltpu.make_async_copy(out_v.at[b], out_ref.at[b],
                                       out_sems.at[b])
            cp.start()
            out_cps.append(cp)
        for cp in out_cps:
            cp.wait()

    call = pl.pallas_call(
        body,
        out_shape=jax.ShapeDtypeStruct((B, S, D), BF),
        in_specs=[pl.BlockSpec(memory_space=pltpu.VMEM)] * 6,
        out_specs=pl.BlockSpec(memory_space=pltpu.HBM),
        scratch_shapes=[
            pltpu.VMEM((T, DC), BF),
            pltpu.VMEM((T, DC), BF),
            pltpu.VMEM((2 * DC, HD), BF),
            pltpu.VMEM((T, D), BF),
            pltpu.VMEM((B, S, D), BF),
            pltpu.SemaphoreType.DMA((2,)),
            pltpu.SemaphoreType.DMA((2,)),
            pltpu.SemaphoreType.DMA((HH,)),
            pltpu.SemaphoreType.DMA((HH,)),
            pltpu.SemaphoreType.DMA((B,)),
        ],
        compiler_params=pltpu.CompilerParams(collective_id=0),
    )
    return call(x.astype(BF),
                jnp.concatenate([Wdkv, Wkr], axis=1).astype(BF),
                jnp.concatenate([Wuk, Wuv], axis=0).astype(BF),
                Wq.astype(BF), Wqr.astype(BF), Wo.astype(BF))
